# Initial kernel scaffold; baseline (speedup 1.0000x reference)
import jax
import jax.numpy as jnp
from jax.experimental import pallas as pl


def kernel(word_emb, user_feature_emb, item_feature_emb, user_word_embedding, item_word_embedding, user_proj_w, user_proj_b, item_proj_w, item_proj_b, w0_w, w0_b, w1_w, w1_b, user_features, item_features, user_name_src, user_name_dst, user_comment_src, user_comment_dst, item_name_src, item_name_dst, item_comment_src, item_comment_dst, edge_index):
    raise NotImplementedError("write your pallas kernel here")


# speedup vs baseline: 4.6345x; 4.6345x over previous
import functools

import jax
import jax.numpy as jnp
from jax import lax
from jax.experimental import pallas as pl
from jax.experimental.pallas import tpu as pltpu
from jax.experimental.pallas import tpu_sc as plsc

N_USER = 30000
M_ITEM = 20000
N_NODES = N_USER + M_ITEM
D = 64
WD = 32

_NTILES = 16
_NCORES = 2
_C = 128

_RB = 512
NU_P = 30208
MI_P = 20480
NN_P = 50176


def _bag_kernel(n_chunks_per_tile, npad, dh, with_counts):
    mesh = plsc.VectorSubcoreMesh(core_axis_name="c", subcore_axis_name="s")
    rows_per_tile = npad // _NTILES
    per_tile = n_chunks_per_tile * _C

    out_type = [jax.ShapeDtypeStruct((_NCORES, npad, dh), jnp.float32)]
    if with_counts:
        out_type.append(jax.ShapeDtypeStruct((npad,), jnp.float32))

    scratch = [
        pltpu.VMEM_SHARED((npad, dh), jnp.float32),
        pltpu.VMEM((_C,), jnp.int32),
        pltpu.VMEM((_C,), jnp.int32),
        pltpu.VMEM((_C, dh), jnp.float32),
        pltpu.SemaphoreType.DMA,
    ]
    if with_counts:
        scratch.append(pltpu.VMEM_SHARED((npad,), jnp.float32))
        scratch.append(pltpu.VMEM((_C,), jnp.float32))

    def body(table2, idx2, dst, zrows, zcnt, *refs):
        if with_counts:
            (sums_out, cnt_out, acc, idx_v, dst_v, rows_v, sem, cnt_acc,
             ones_v) = refs
        else:
            sums_out, acc, idx_v, dst_v, rows_v, sem = refs
        cid = lax.axis_index("c")
        sid = lax.axis_index("s")
        base_r = sid * rows_per_tile

        pltpu.sync_copy(zrows.at[pl.ds(base_r, rows_per_tile)],
                        acc.at[pl.ds(base_r, rows_per_tile)])
        if with_counts:
            pltpu.sync_copy(zcnt.at[pl.ds(base_r, rows_per_tile)],
                            cnt_acc.at[pl.ds(base_r, rows_per_tile)])
            for j in range(_C // 16):
                ones_v[pl.ds(j * 16, 16)] = jnp.ones((16,), jnp.float32)
        plsc.subcore_barrier()

        def chunk(i, _):
            base = sid * per_tile + i * _C
            pltpu.sync_copy(idx2.at[cid, pl.ds(base, _C)], idx_v)
            pltpu.sync_copy(dst.at[pl.ds(base, _C)], dst_v)
            pltpu.async_copy(table2.at[idx_v], rows_v, sem).wait()
            pltpu.sync_copy(rows_v, acc.at[dst_v], add=True)
            if with_counts:
                @pl.when(cid == 0)
                def _():
                    pltpu.sync_copy(ones_v, cnt_acc.at[dst_v], add=True)
            return _

        lax.fori_loop(0, n_chunks_per_tile, chunk, None)
        plsc.subcore_barrier()

        pltpu.sync_copy(acc.at[pl.ds(base_r, rows_per_tile)],
                        sums_out.at[cid, pl.ds(base_r, rows_per_tile)])
        if with_counts:
            @pl.when(cid == 0)
            def _():
                pltpu.sync_copy(cnt_acc.at[pl.ds(base_r, rows_per_tile)],
                                cnt_out.at[pl.ds(base_r, rows_per_tile)])

    return pl.kernel(body, out_type=tuple(out_type), mesh=mesh,
                     scratch_types=scratch,
                     compiler_params=pltpu.CompilerParams(
                         use_tc_tiling_on_sc=False))


def _bag(table, src, dst, npad, dh, with_counts):
    ne = src.shape[0]
    ne_pad = -(-ne // (_NTILES * _C)) * (_NTILES * _C)
    n_chunks = ne_pad // (_NTILES * _C)
    pad = ne_pad - ne
    src = jnp.concatenate([src, jnp.zeros((pad,), src.dtype)]).astype(jnp.int32)
    dst = jnp.concatenate(
        [dst.astype(jnp.int32), jnp.full((pad,), npad - 1, jnp.int32)])
    idx2 = jnp.stack([2 * src, 2 * src + 1])
    table2 = table.reshape(table.shape[0] * 2, dh)
    zrows = jnp.zeros((npad, dh), jnp.float32)
    zcnt = jnp.zeros((npad,), jnp.float32)
    fn = _bag_kernel(n_chunks, npad, dh, with_counts)
    out = fn(table2, idx2, dst, zrows, zcnt)
    if with_counts:
        return out[0], out[1]
    return out[0], None


def _proj_tc(name_s, name_c, com_s, com_c, we, feat_s, w, b, n_rows):
    npad = name_s.shape[1]
    grid = n_rows // _RB if n_rows % _RB == 0 else -(-n_rows // _RB)
    wn0, wn1 = w[0:16], w[16:32]
    wc0, wc1 = w[32:48], w[48:64]
    wwe = w[64:364]
    wf0, wf1 = w[364:396], w[396:428]
    name_c = name_c.reshape(npad, 1)
    com_c = com_c.reshape(npad, 1)

    def body(ns, nc, cs, cc, we_r, fs, wn0_r, wn1_r, wc0_r, wc1_r, wwe_r,
             wf0_r, wf1_r, b_r, out):
        inv_n = 1.0 / jnp.maximum(nc[...], 1.0)
        inv_c = 1.0 / jnp.maximum(cc[...], 1.0)
        f32 = jnp.float32
        acc = jnp.dot(ns[0] * inv_n, wn0_r[...], preferred_element_type=f32)
        acc += jnp.dot(ns[1] * inv_n, wn1_r[...], preferred_element_type=f32)
        acc += jnp.dot(cs[0] * inv_c, wc0_r[...], preferred_element_type=f32)
        acc += jnp.dot(cs[1] * inv_c, wc1_r[...], preferred_element_type=f32)
        acc += jnp.dot(we_r[...], wwe_r[...], preferred_element_type=f32)
        acc += jnp.dot(fs[0] * 0.1, wf0_r[...], preferred_element_type=f32)
        acc += jnp.dot(fs[1] * 0.1, wf1_r[...], preferred_element_type=f32)
        out[...] = acc + b_r[...]

    whole = lambda shape: pl.BlockSpec(shape, lambda i: (0,) * len(shape))
    return pl.pallas_call(
        body,
        grid=(grid,),
        in_specs=[
            pl.BlockSpec((2, _RB, 16), lambda i: (0, i, 0)),
            pl.BlockSpec((_RB, 1), lambda i: (i, 0)),
            pl.BlockSpec((2, _RB, 16), lambda i: (0, i, 0)),
            pl.BlockSpec((_RB, 1), lambda i: (i, 0)),
            pl.BlockSpec((_RB, 300), lambda i: (i, 0)),
            pl.BlockSpec((2, _RB, 32), lambda i: (0, i, 0)),
            whole((16, D)), whole((16, D)), whole((16, D)), whole((16, D)),
            whole((300, D)), whole((32, D)), whole((32, D)), whole((1, D)),
        ],
        out_specs=pl.BlockSpec((_RB, D), lambda i: (i, 0)),
        out_shape=jax.ShapeDtypeStruct((n_rows, D), jnp.float32),
    )(name_s, name_c, com_s, com_c, we, feat_s, wn0, wn1, wc0, wc1, wwe,
      wf0, wf1, b.reshape(1, D))


def _layer_tc(x, s, c, w, b, relu):
    n = x.shape[0]
    npad = s.shape[1]
    grid = -(-n // _RB)
    wx, wa0, wa1 = w[0:64], w[64:96], w[96:128]
    c = c.reshape(npad, 1)

    def body(x_r, s_r, c_r, wx_r, wa0_r, wa1_r, b_r, out):
        inv = 1.0 / jnp.maximum(c_r[...], 1.0)
        f32 = jnp.float32
        acc = jnp.dot(x_r[...], wx_r[...], preferred_element_type=f32)
        acc += jnp.dot(s_r[0] * inv, wa0_r[...], preferred_element_type=f32)
        acc += jnp.dot(s_r[1] * inv, wa1_r[...], preferred_element_type=f32)
        acc += b_r[...]
        if relu:
            acc = jnp.maximum(acc, 0.0)
        out[...] = acc

    whole = lambda shape: pl.BlockSpec(shape, lambda i: (0,) * len(shape))
    return pl.pallas_call(
        body,
        grid=(grid,),
        in_specs=[
            pl.BlockSpec((_RB, D), lambda i: (i, 0)),
            pl.BlockSpec((2, _RB, 32), lambda i: (0, i, 0)),
            pl.BlockSpec((_RB, 1), lambda i: (i, 0)),
            whole((64, D)), whole((32, D)), whole((32, D)), whole((1, D)),
        ],
        out_specs=pl.BlockSpec((_RB, D), lambda i: (i, 0)),
        out_shape=jax.ShapeDtypeStruct((n, D), jnp.float32),
    )(x, s, c, wx, wa0, wa1, b.reshape(1, D))


def kernel(word_emb, user_feature_emb, item_feature_emb, user_word_embedding,
           item_word_embedding, user_proj_w, user_proj_b, item_proj_w,
           item_proj_b, w0_w, w0_b, w1_w, w1_b, user_features, item_features,
           user_name_src, user_name_dst, user_comment_src, user_comment_dst,
           item_name_src, item_name_dst, item_comment_src, item_comment_dst,
           edge_index):
    un_s, un_c = _bag(word_emb, user_name_src, user_name_dst, NU_P, 16, True)
    uc_s, uc_c = _bag(word_emb, user_comment_src, user_comment_dst, NU_P, 16,
                      True)
    in_s, in_c = _bag(word_emb, item_name_src, item_name_dst, MI_P, 16, True)
    ic_s, ic_c = _bag(word_emb, item_comment_src, item_comment_dst, MI_P, 16,
                      True)

    uf_dst = jnp.arange(N_USER * 10, dtype=jnp.int32) // 10
    if_dst = jnp.arange(M_ITEM * 10, dtype=jnp.int32) // 10
    uf_s, _ = _bag(user_feature_emb, user_features.reshape(-1), uf_dst,
                   NU_P, 32, False)
    if_s, _ = _bag(item_feature_emb, item_features.reshape(-1), if_dst,
                   MI_P, 32, False)

    u_init = _proj_tc(un_s, un_c, uc_s, uc_c, user_word_embedding, uf_s,
                      user_proj_w, user_proj_b, N_USER)
    i_init = _proj_tc(in_s, in_c, ic_s, ic_c, item_word_embedding, if_s,
                      item_proj_w, item_proj_b, M_ITEM)
    x = jnp.concatenate([u_init, i_init], axis=0)

    src = edge_index[0]
    dst = edge_index[1]
    s0, e_c = _bag(x, src, dst, NN_P, 32, True)
    x = _layer_tc(x, s0, e_c, w0_w, w0_b, relu=True)
    s1, _ = _bag(x, src, dst, NN_P, 32, False)
    x = _layer_tc(x, s1, e_c, w1_w, w1_b, relu=False)
    return x


# speedup vs baseline: 10.4283x; 2.2501x over previous
import functools

import jax
import jax.numpy as jnp
from jax import lax
from jax.experimental import pallas as pl
from jax.experimental.pallas import tpu as pltpu
from jax.experimental.pallas import tpu_sc as plsc

N_USER = 30000
M_ITEM = 20000
N_NODES = N_USER + M_ITEM
D = 64
WD = 32

_NTILES = 16
_NCORES = 2
_C = 128
_S = 8

_RB = 512
NU_P = 30208
MI_P = 20480
NN_P = 50176


def _bag_kernel(n_super, npad, dh, with_counts, s_chunks):
    mesh = plsc.VectorSubcoreMesh(core_axis_name="c", subcore_axis_name="s")
    rows_per_tile = npad // _NTILES
    per_tile_chunks = n_super * s_chunks
    n_triples = -(-n_super // 3)

    out_type = [jax.ShapeDtypeStruct((_NCORES, npad, dh), jnp.float32)]
    if with_counts:
        out_type.append(jax.ShapeDtypeStruct((_NCORES, npad), jnp.float32))

    scratch = [
        pltpu.VMEM_SHARED((npad, dh), jnp.float32),
        pltpu.VMEM((3, s_chunks, _C), jnp.int32),
        pltpu.VMEM((3, s_chunks, _C), jnp.int32),
        pltpu.VMEM((3, s_chunks, _C, dh), jnp.float32),
        pltpu.SemaphoreType.DMA,
        pltpu.SemaphoreType.DMA, pltpu.SemaphoreType.DMA,
        pltpu.SemaphoreType.DMA,
        pltpu.SemaphoreType.DMA, pltpu.SemaphoreType.DMA,
        pltpu.SemaphoreType.DMA,
    ]
    if with_counts:
        scratch += [
            pltpu.VMEM_SHARED((npad,), jnp.float32),
            pltpu.VMEM((_C,), jnp.float32),
            pltpu.SemaphoreType.DMA, pltpu.SemaphoreType.DMA,
            pltpu.SemaphoreType.DMA,
        ]

    def body(table2, idx2, dsth, zrows, zcnt, *refs):
        if with_counts:
            (sums_out, cnt_out, acc, idx_s, dst_s, rows_v, g_sem,
             p0, p1, p2, q0, q1, q2, cnt_acc, ones_v, c0, c1, c2) = refs
            c_sems = (c0, c1, c2)
        else:
            (sums_out, acc, idx_s, dst_s, rows_v, g_sem,
             p0, p1, p2, q0, q1, q2) = refs
        sup_sems = (p0, p1, p2)
        s_sems = (q0, q1, q2)
        cid = lax.axis_index("c")
        sid = lax.axis_index("s")
        base_r = sid * rows_per_tile
        chunk_base = sid * per_tile_chunks

        pltpu.sync_copy(zrows.at[pl.ds(base_r, rows_per_tile)],
                        acc.at[pl.ds(base_r, rows_per_tile)])
        if with_counts:
            pltpu.sync_copy(zcnt.at[pl.ds(base_r, rows_per_tile)],
                            cnt_acc.at[pl.ds(base_r, rows_per_tile)])
            for j in range(_C // 16):
                ones_v[pl.ds(j * 16, 16)] = jnp.ones((16,), jnp.float32)
        plsc.subcore_barrier()

        def sup_load(p, st, issue):
            a = pltpu.make_async_copy(
                idx2.at[cid, pl.ds(chunk_base + p * s_chunks, s_chunks)], idx_s.at[st],
                sup_sems[st])
            b = pltpu.make_async_copy(
                dsth.at[pl.ds(chunk_base + p * s_chunks, s_chunks)], dst_s.at[st],
                sup_sems[st])
            if issue:
                a.start()
                b.start()
            else:
                a.wait()
                b.wait()

        def drain_set(st):
            for j in range(s_chunks):
                pltpu.make_async_copy(rows_v.at[st, j],
                                      acc.at[dst_s.at[st, j]],
                                      s_sems[st]).wait()
            if with_counts:
                for j in range(s_chunks):
                    @pl.when(cid == (j % 2))
                    def _():
                        pltpu.make_async_copy(ones_v,
                                              cnt_acc.at[dst_s.at[st, j]],
                                              c_sems[st]).wait()

        def process(p, st, s2):
            @pl.when(p >= 2)
            def _():
                drain_set(s2)

            @pl.when(p + 1 < n_super)
            def _():
                sup_load(p + 1, s2, issue=True)

            sup_load(p, st, issue=False)
            for j in range(s_chunks):
                pltpu.async_copy(table2.at[idx_s.at[st, j]],
                                 rows_v.at[st, j], g_sem)
            for j in range(s_chunks):
                pltpu.make_async_copy(table2.at[idx_s.at[st, j]],
                                      rows_v.at[st, j], g_sem).wait()
            for j in range(s_chunks):
                pltpu.async_copy(rows_v.at[st, j], acc.at[dst_s.at[st, j]],
                                 s_sems[st], add=True)
            if with_counts:
                for j in range(s_chunks):
                    @pl.when(cid == (j % 2))
                    def _():
                        pltpu.async_copy(ones_v, cnt_acc.at[dst_s.at[st, j]],
                                         c_sems[st], add=True)

        sup_load(0, 0, issue=True)

        def triple(q, _):
            for k in range(3):
                p = 3 * q + k

                @pl.when(p < n_super)
                def _():
                    process(p, k, (k + 1) % 3)
            return _

        lax.fori_loop(0, n_triples, triple, None)
        for p in range(max(n_super - 2, 0), n_super):
            drain_set(p % 3)
        plsc.subcore_barrier()

        pltpu.sync_copy(acc.at[pl.ds(base_r, rows_per_tile)],
                        sums_out.at[cid, pl.ds(base_r, rows_per_tile)])
        if with_counts:
            pltpu.sync_copy(cnt_acc.at[pl.ds(base_r, rows_per_tile)],
                            cnt_out.at[cid, pl.ds(base_r, rows_per_tile)])

    return pl.kernel(body, out_type=tuple(out_type), mesh=mesh,
                     scratch_types=scratch,
                     compiler_params=pltpu.CompilerParams(
                         use_tc_tiling_on_sc=False))


def _bag(table, src, dst, npad, dh, with_counts):
    ne = src.shape[0]
    acc_words = npad * dh + (npad if with_counts else 0)
    per_tile = (2097151 - acc_words) // _NTILES
    s_chunks = 1
    for cand in (8, 4, 2):
        if 3 * cand * _C * dh + 6 * cand * _C + 300 <= per_tile:
            s_chunks = cand
            break
    sup_edges = _NTILES * _C * s_chunks
    n_super = -(-ne // sup_edges)
    ne_pad = n_super * sup_edges
    pad = ne_pad - ne
    src = jnp.concatenate([src, jnp.zeros((pad,), src.dtype)]).astype(jnp.int32)
    dst = jnp.concatenate(
        [dst.astype(jnp.int32), jnp.full((pad,), npad - 1, jnp.int32)])
    idx2 = jnp.stack([2 * src, 2 * src + 1]).reshape(2, ne_pad // _C, _C)
    dsth = dst.reshape(ne_pad // _C, _C)
    table2 = table.reshape(table.shape[0] * 2, dh)
    zrows = jnp.zeros((npad, dh), jnp.float32)
    zcnt = jnp.zeros((npad,), jnp.float32)
    fn = _bag_kernel(n_super, npad, dh, with_counts, s_chunks)
    out = fn(table2, idx2, dsth, zrows, zcnt)
    if with_counts:
        return out[0], out[1]
    return out[0], None


def _proj_tc(name_s, name_c, com_s, com_c, we, feat_s, w, b, n_rows):
    grid = -(-n_rows // _RB)
    wn0, wn1 = w[0:16], w[16:32]
    wc0, wc1 = w[32:48], w[48:64]
    wwe = w[64:364]
    wf0, wf1 = w[364:396], w[396:428]
    name_c = name_c[..., None]
    com_c = com_c[..., None]

    def body(ns, nc, cs, cc, we_r, fs, wn0_r, wn1_r, wc0_r, wc1_r, wwe_r,
             wf0_r, wf1_r, b_r, out):
        inv_n = 1.0 / jnp.maximum(nc[0] + nc[1], 1.0)
        inv_c = 1.0 / jnp.maximum(cc[0] + cc[1], 1.0)
        f32 = jnp.float32
        acc = jnp.dot(ns[0] * inv_n, wn0_r[...], preferred_element_type=f32)
        acc += jnp.dot(ns[1] * inv_n, wn1_r[...], preferred_element_type=f32)
        acc += jnp.dot(cs[0] * inv_c, wc0_r[...], preferred_element_type=f32)
        acc += jnp.dot(cs[1] * inv_c, wc1_r[...], preferred_element_type=f32)
        acc += jnp.dot(we_r[...], wwe_r[...], preferred_element_type=f32)
        acc += jnp.dot(fs[0] * 0.1, wf0_r[...], preferred_element_type=f32)
        acc += jnp.dot(fs[1] * 0.1, wf1_r[...], preferred_element_type=f32)
        out[...] = acc + b_r[...]

    whole = lambda shape: pl.BlockSpec(shape, lambda i: (0,) * len(shape))
    return pl.pallas_call(
        body,
        grid=(grid,),
        in_specs=[
            pl.BlockSpec((2, _RB, 16), lambda i: (0, i, 0)),
            pl.BlockSpec((2, _RB, 1), lambda i: (0, i, 0)),
            pl.BlockSpec((2, _RB, 16), lambda i: (0, i, 0)),
            pl.BlockSpec((2, _RB, 1), lambda i: (0, i, 0)),
            pl.BlockSpec((_RB, 300), lambda i: (i, 0)),
            pl.BlockSpec((2, _RB, 32), lambda i: (0, i, 0)),
            whole((16, D)), whole((16, D)), whole((16, D)), whole((16, D)),
            whole((300, D)), whole((32, D)), whole((32, D)), whole((1, D)),
        ],
        out_specs=pl.BlockSpec((_RB, D), lambda i: (i, 0)),
        out_shape=jax.ShapeDtypeStruct((n_rows, D), jnp.float32),
    )(name_s, name_c, com_s, com_c, we, feat_s, wn0, wn1, wc0, wc1, wwe,
      wf0, wf1, b.reshape(1, D))


def _layer_tc(x, s, c, w, b, relu):
    n = x.shape[0]
    grid = -(-n // _RB)
    wx, wa0, wa1 = w[0:64], w[64:96], w[96:128]
    c = c[..., None]

    def body(x_r, s_r, c_r, wx_r, wa0_r, wa1_r, b_r, out):
        inv = 1.0 / jnp.maximum(c_r[0] + c_r[1], 1.0)
        f32 = jnp.float32
        acc = jnp.dot(x_r[...], wx_r[...], preferred_element_type=f32)
        acc += jnp.dot(s_r[0] * inv, wa0_r[...], preferred_element_type=f32)
        acc += jnp.dot(s_r[1] * inv, wa1_r[...], preferred_element_type=f32)
        acc += b_r[...]
        if relu:
            acc = jnp.maximum(acc, 0.0)
        out[...] = acc

    whole = lambda shape: pl.BlockSpec(shape, lambda i: (0,) * len(shape))
    return pl.pallas_call(
        body,
        grid=(grid,),
        in_specs=[
            pl.BlockSpec((_RB, D), lambda i: (i, 0)),
            pl.BlockSpec((2, _RB, 32), lambda i: (0, i, 0)),
            pl.BlockSpec((2, _RB, 1), lambda i: (0, i, 0)),
            whole((64, D)), whole((32, D)), whole((32, D)), whole((1, D)),
        ],
        out_specs=pl.BlockSpec((_RB, D), lambda i: (i, 0)),
        out_shape=jax.ShapeDtypeStruct((n, D), jnp.float32),
    )(x, s, c, wx, wa0, wa1, b.reshape(1, D))


def kernel(word_emb, user_feature_emb, item_feature_emb, user_word_embedding,
           item_word_embedding, user_proj_w, user_proj_b, item_proj_w,
           item_proj_b, w0_w, w0_b, w1_w, w1_b, user_features, item_features,
           user_name_src, user_name_dst, user_comment_src, user_comment_dst,
           item_name_src, item_name_dst, item_comment_src, item_comment_dst,
           edge_index):
    un_s, un_c = _bag(word_emb, user_name_src, user_name_dst, NU_P, 16, True)
    uc_s, uc_c = _bag(word_emb, user_comment_src, user_comment_dst, NU_P, 16,
                      True)
    in_s, in_c = _bag(word_emb, item_name_src, item_name_dst, MI_P, 16, True)
    ic_s, ic_c = _bag(word_emb, item_comment_src, item_comment_dst, MI_P, 16,
                      True)

    uf_dst = jnp.arange(N_USER * 10, dtype=jnp.int32) // 10
    if_dst = jnp.arange(M_ITEM * 10, dtype=jnp.int32) // 10
    uf_s, _ = _bag(user_feature_emb, user_features.reshape(-1), uf_dst,
                   NU_P, 32, False)
    if_s, _ = _bag(item_feature_emb, item_features.reshape(-1), if_dst,
                   MI_P, 32, False)

    u_init = _proj_tc(un_s, un_c, uc_s, uc_c, user_word_embedding, uf_s,
                      user_proj_w, user_proj_b, N_USER)
    i_init = _proj_tc(in_s, in_c, ic_s, ic_c, item_word_embedding, if_s,
                      item_proj_w, item_proj_b, M_ITEM)
    x = jnp.concatenate([u_init, i_init], axis=0)

    src = edge_index[0]
    dst = edge_index[1]
    s0, e_c = _bag(x, src, dst, NN_P, 32, True)
    x = _layer_tc(x, s0, e_c, w0_w, w0_b, relu=True)
    s1, _ = _bag(x, src, dst, NN_P, 32, False)
    x = _layer_tc(x, s1, e_c, w1_w, w1_b, relu=False)
    return x
